# baseline (speedup 1.0000x reference)
import functools

import jax
import jax.numpy as jnp
from jax import lax
from jax.experimental import pallas as pl
from jax.experimental.pallas import tpu as pltpu
from jax.experimental.pallas import tpu_sc as plsc

B = 320000
D = 128
L = 16
NC, NS = 2, 16
NW = NC * NS
B_PER_W = B // NW
CHUNK = 400
NCHUNK = B_PER_W // CHUNK
NGROUP = CHUNK // L


def _body(emb1_hbm, emb2_hbm, idx_src_hbm, idx_dst_hbm, out_hbm,
          idx_s_v, idx_d_v, src_v, dst_v, out_v, sem_s, sem_d):
    wid = lax.axis_index("s") * NC + lax.axis_index("c")
    base_w = wid * B_PER_W
    lane = lax.iota(jnp.int32, L)

    def chunk_body(i, carry):
        base = base_w + i * CHUNK
        pltpu.sync_copy(idx_src_hbm.at[pl.ds(base, CHUNK)], idx_s_v)
        pltpu.sync_copy(idx_dst_hbm.at[pl.ds(base, CHUNK)], idx_d_v)
        cp_s = pltpu.async_copy(emb1_hbm.at[idx_s_v], src_v, sem_s)
        cp_d = pltpu.async_copy(emb2_hbm.at[idx_d_v], dst_v, sem_d)
        cp_s.wait()
        cp_d.wait()

        for g in range(NGROUP):
            rows = lane + (g * L)

            def d_body(i, accs):
                cb = jnp.zeros((L,), jnp.int32) + (i * 8)
                new = []
                for k in range(8):
                    col = cb + k
                    sv = plsc.load_gather(src_v, [rows, col])
                    dv = plsc.load_gather(dst_v, [rows, col])
                    new.append(accs[k] + sv * dv)
                return tuple(new)

            accs = lax.fori_loop(
                0, D // 8, d_body,
                tuple(jnp.zeros((L,), jnp.float32) for _ in range(8)))
            a0 = (accs[0] + accs[1]) + (accs[2] + accs[3])
            a1 = (accs[4] + accs[5]) + (accs[6] + accs[7])
            out_v[pl.ds(g * L, L)] = a0 + a1

        pltpu.sync_copy(out_v, out_hbm.at[pl.ds(base, CHUNK)])
        return carry

    lax.fori_loop(0, NCHUNK, chunk_body, 0)


@jax.jit
def _classify(emb1, emb2, idx_src, idx_dst):
    mesh = plsc.VectorSubcoreMesh(core_axis_name="c", subcore_axis_name="s")
    return pl.kernel(
        _body,
        out_type=jax.ShapeDtypeStruct((B,), jnp.float32),
        mesh=mesh,
        scratch_types=[
            pltpu.VMEM((CHUNK,), jnp.int32),
            pltpu.VMEM((CHUNK,), jnp.int32),
            pltpu.VMEM((CHUNK, D), jnp.float32),
            pltpu.VMEM((CHUNK, D), jnp.float32),
            pltpu.VMEM((CHUNK,), jnp.float32),
            pltpu.SemaphoreType.DMA,
            pltpu.SemaphoreType.DMA,
        ],
        compiler_params=pltpu.CompilerParams(needs_layout_passes=False),
    )(emb1, emb2, idx_src, idx_dst)


def kernel(embedding_1, embedding_2, edge_label_index):
    idx = edge_label_index.astype(jnp.int32)
    return _classify(embedding_1, embedding_2, idx[0], idx[1])


# speedup vs baseline: 3.1364x; 3.1364x over previous
import functools

import jax
import jax.numpy as jnp
from jax import lax
from jax.experimental import pallas as pl
from jax.experimental.pallas import tpu as pltpu
from jax.experimental.pallas import tpu_sc as plsc

B = 320000
D = 128
L = 16
NC, NS = 2, 16
NW = NC * NS
B_PER_W = B // NW
CHUNK = 200
NCHUNK = B_PER_W // CHUNK
NGROUP = CHUNK // L


def _body(emb1_hbm, emb2_hbm, idx_src_hbm, idx_dst_hbm, out_hbm,
          idx_s_v, idx_d_v, src_v, dst_v, part_v, out_v, sem_s, sem_d):
    wid = lax.axis_index("s") * NC + lax.axis_index("c")
    base_w = wid * B_PER_W
    lane = lax.iota(jnp.int32, L)

    def chunk_body(i, carry):
        base = base_w + i * CHUNK
        pltpu.sync_copy(idx_src_hbm.at[pl.ds(base, CHUNK)], idx_s_v)
        pltpu.sync_copy(idx_dst_hbm.at[pl.ds(base, CHUNK)], idx_d_v)
        cp_s = pltpu.async_copy(emb1_hbm.at[idx_s_v], src_v, sem_s)
        cp_d = pltpu.async_copy(emb2_hbm.at[idx_d_v], dst_v, sem_d)
        cp_s.wait()
        cp_d.wait()

        def e_body(e, carry):
            s_row = src_v.at[e]
            d_row = dst_v.at[e]
            parts = []
            for k in range(D // L):
                sv = s_row[pl.ds(k * L, L)]
                dv = d_row[pl.ds(k * L, L)]
                parts.append(sv * dv)
            while len(parts) > 1:
                parts = [a + b for a, b in zip(parts[::2], parts[1::2])]
            part_v.at[e][pl.ds(0, L)] = parts[0]
            return carry

        lax.fori_loop(0, CHUNK, e_body, 0, unroll=2)

        for g in range(NGROUP):
            rows = lane + (g * L)
            accs = [jnp.zeros((L,), jnp.float32) for _ in range(4)]
            for c in range(L):
                col = jnp.zeros((L,), jnp.int32) + c
                accs[c % 4] = accs[c % 4] + plsc.load_gather(
                    part_v, [rows, col])
            out_v[pl.ds(g * L, L)] = (accs[0] + accs[1]) + (accs[2] + accs[3])

        pltpu.sync_copy(out_v, out_hbm.at[pl.ds(base, CHUNK)])
        return carry

    lax.fori_loop(0, NCHUNK, chunk_body, 0)


@jax.jit
def _classify(emb1, emb2, idx_src, idx_dst):
    mesh = plsc.VectorSubcoreMesh(core_axis_name="c", subcore_axis_name="s")
    return pl.kernel(
        _body,
        out_type=jax.ShapeDtypeStruct((B,), jnp.float32),
        mesh=mesh,
        scratch_types=[
            pltpu.VMEM((CHUNK,), jnp.int32),
            pltpu.VMEM((CHUNK,), jnp.int32),
            pltpu.VMEM((CHUNK, D), jnp.float32),
            pltpu.VMEM((CHUNK, D), jnp.float32),
            pltpu.VMEM((CHUNK, L + 1), jnp.float32),
            pltpu.VMEM((CHUNK,), jnp.float32),
            pltpu.SemaphoreType.DMA,
            pltpu.SemaphoreType.DMA,
        ],
        compiler_params=pltpu.CompilerParams(needs_layout_passes=False),
    )(emb1, emb2, idx_src, idx_dst)


def kernel(embedding_1, embedding_2, edge_label_index):
    idx = edge_label_index.astype(jnp.int32)
    return _classify(embedding_1, embedding_2, idx[0], idx[1])
